# baseline (speedup 1.0000x reference)
import functools

import jax
import jax.numpy as jnp
import numpy as np
from jax import lax
from jax.experimental import pallas as pl
from jax.experimental.pallas import tpu as pltpu
from jax.experimental.pallas import tpu_sc as plsc

B, H, W, C = 16, 224, 224, 96
CP = 128
HW = H * W
L = 16
NTILES = 32
CHUNK = 64
CPI = HW // CHUNK
NCHUNK = B * CPI // NTILES
NGRP = CHUNK // L
CROW = C // L
DELTA = float(np.float32(2.0) / np.float32(W - 1))


def _bf16r(x):
    u = lax.bitcast_convert_type(x, jnp.int32)
    u = u + 0x7FFF + ((u >> 16) & 1)
    u = lax.bitwise_and(u, jnp.int32(-65536))
    return lax.bitcast_convert_type(u, jnp.float32)


def _tile_body(coef_hbm, table_hbm, out_hbm, coef_v,
               ia0, ib0, ic0, id0, ia1, ib1, ic1, id1,
               w0, w1, ga0, gb0, gc0, gd0, ga1, gb1, gc1, gd1,
               o0, o1, zb, flag_v, gsem0, gsem1, osem0, osem1):
    idx_refs = ((ia0, ib0, ic0, id0), (ia1, ib1, ic1, id1))
    g_refs = ((ga0, gb0, gc0, gd0), (ga1, gb1, gc1, gd1))
    w_refs = (w0, w1)
    o_refs = (o0, o1)
    gsems = (gsem0, gsem1)
    osems = (osem0, osem1)
    wid = lax.axis_index("s") * 2 + lax.axis_index("c")

    pltpu.sync_copy(coef_hbm, coef_v)
    lane = lax.broadcasted_iota(jnp.int32, (L,), 0)

    def zero_grp(g, _):
        z = jnp.zeros((L,), jnp.float32)
        for r in range(CROW):
            zb[pl.ds(g * C + r * L, L)] = z
        return _

    lax.fori_loop(0, CHUNK, zero_grp, None, unroll=False)

    def compute_idx(cg, s):
        ia_v, ib_v, ic_v, id_v = idx_refs[s]
        w_v = w_refs[s]
        b = cg // CPI
        pid0 = (cg % CPI) * CHUNK
        imgbase = b * HW
        ax = _bf16r(coef_v[b * 6 + 0])
        bx = _bf16r(coef_v[b * 6 + 1])
        cx = _bf16r(coef_v[b * 6 + 2])
        ay = _bf16r(coef_v[b * 6 + 3])
        by = _bf16r(coef_v[b * 6 + 4])
        cy = _bf16r(coef_v[b * 6 + 5])

        def idx_grp(g, acc):
            pid = pid0 + g * L + lane
            fpid = pid.astype(jnp.float32)
            iv = (fpid * (1.0 / W)).astype(jnp.int32)
            iv = jnp.where(iv * W > pid, iv - 1, iv)
            iv = jnp.where((iv + 1) * W <= pid, iv + 1, iv)
            jv = pid - iv * W
            fj = jv.astype(jnp.float32)
            fiv = iv.astype(jnp.float32)
            xtv = _bf16r(fj * DELTA + (-1.0))
            ytv = _bf16r(fiv * DELTA + (-1.0))
            sgx = ax * xtv + bx * ytv + cx
            sgy = ay * xtv + by * ytv + cy
            xf = (sgx + 1.0) * (W * 0.5)
            yf = (sgy + 1.0) * (H * 0.5)
            xt = xf.astype(jnp.int32).astype(jnp.float32)
            x0f = jnp.where(xt > xf, xt - 1.0, xt)
            yt = yf.astype(jnp.int32).astype(jnp.float32)
            y0f = jnp.where(yt > yf, yt - 1.0, yt)
            x0c = jnp.clip(x0f, 0.0, W - 1.0)
            x1c = jnp.clip(x0f + 1.0, 0.0, W - 1.0)
            y0c = jnp.clip(y0f, 0.0, H - 1.0)
            y1c = jnp.clip(y0f + 1.0, 0.0, H - 1.0)
            dx1 = x1c - xf
            dx0 = xf - x0c
            dy1 = y1c - yf
            dy0 = yf - y0c
            x0i = x0c.astype(jnp.int32)
            y0i = y0c.astype(jnp.int32)
            dxs = x1c.astype(jnp.int32) - x0i
            dys = (y1c.astype(jnp.int32) - y0i) * W
            ia = imgbase + y0i * W + x0i
            sl = pl.ds(g * L, L)
            ia_v[sl] = ia
            ib_v[sl] = ia + dys
            ic_v[sl] = ia + dxs
            id_v[sl] = ia + dys + dxs
            w_v[pl.ds(0 * CHUNK + g * L, L)] = dx1 * dy1
            w_v[pl.ds(1 * CHUNK + g * L, L)] = dx1 * dy0
            w_v[pl.ds(2 * CHUNK + g * L, L)] = dx0 * dy1
            w_v[pl.ds(3 * CHUNK + g * L, L)] = dx0 * dy0
            okx = jnp.logical_and(xf >= 0.0, xf < W - 1.0)
            oky = jnp.logical_and(yf >= 0.0, yf < H - 1.0)
            ok = jnp.where(jnp.logical_and(okx, oky), 1.0, 0.0)
            return jnp.maximum(acc, ok)

        acc = lax.fori_loop(0, NGRP, idx_grp, jnp.zeros((L,), jnp.float32),
                            unroll=False)
        tot = acc[0]
        for p in range(1, L):
            tot = jnp.maximum(tot, acc[p])
        flag_v[s] = (tot > 0.0).astype(jnp.int32)

    def gather_copies(s):
        return [
            pltpu.make_async_copy(table_hbm.at[idx_refs[s][t]],
                                  g_refs[s][t], gsems[s])
            for t in range(4)
        ]

    def out_copy(cg, s):
        return pltpu.make_async_copy(
            o_refs[s], out_hbm.at[pl.ds(cg * (CHUNK * C), CHUNK * C)], osems[s])

    def zero_copy(cg, s):
        return pltpu.make_async_copy(
            zb, out_hbm.at[pl.ds(cg * (CHUNK * C), CHUNK * C)], osems[s])

    def blend(s):
        g_a, g_b, g_c, g_d = g_refs[s]
        w_v = w_refs[s]
        o_v = o_refs[s]

        def blend_grp(g, _):
            wrows = [w_v[pl.ds(t * CHUNK + g * L, L)] for t in range(4)]
            for p in range(L):
                pp = g * L + p
                wv = [jnp.zeros((L,), jnp.float32) + wrows[t][p]
                      for t in range(4)]
                for r in range(CROW):
                    cs = pl.ds(r * L, L)
                    acc = wv[0] * g_a[pp, cs] + wv[1] * g_b[pp, cs]
                    acc = acc + wv[2] * g_c[pp, cs]
                    acc = acc + wv[3] * g_d[pp, cs]
                    o_v[pl.ds(pp * C + r * L, L)] = acc
            return _

        lax.fori_loop(0, NGRP, blend_grp, None, unroll=False)

    def chunk_of(t):
        return wid + t * NTILES

    compute_idx(chunk_of(jnp.int32(0)), 0)

    @pl.when(flag_v[0] != 0)
    def _fire0():
        for cp in gather_copies(0):
            cp.start()

    def pair_step(k, _):
        for s in range(2):
            t = 2 * k + s
            nxt = 1 - s

            @pl.when(t + 1 < NCHUNK)
            def _fire_next():
                compute_idx(chunk_of(t + 1), nxt)

                @pl.when(flag_v[nxt] != 0)
                def _start_next():
                    for cp in gather_copies(nxt):
                        cp.start()

            valid = flag_v[s] != 0

            @pl.when(valid)
            def _drain_gathers():
                for cp in gather_copies(s):
                    cp.wait()

            @pl.when(t >= 2)
            def _drain_out():
                out_copy(chunk_of(t - 2), s).wait()

            @pl.when(valid)
            def _do_blend():
                blend(s)
                out_copy(chunk_of(t), s).start()

            @pl.when(jnp.logical_not(valid))
            def _do_zero():
                zero_copy(chunk_of(t), s).start()
        return _

    lax.fori_loop(0, NCHUNK // 2, pair_step, None, unroll=False)
    out_copy(chunk_of(jnp.int32(NCHUNK - 2)), 0).wait()
    out_copy(chunk_of(jnp.int32(NCHUNK - 1)), 1).wait()


_sc_call = functools.partial(
    pl.kernel,
    mesh=plsc.VectorSubcoreMesh(core_axis_name="c", subcore_axis_name="s"),
    out_type=jax.ShapeDtypeStruct((B * HW * C,), jnp.float32),
    scratch_types=(
        [pltpu.VMEM((B * 6, L), jnp.float32)]
        + [pltpu.VMEM((CHUNK,), jnp.int32)] * 8
        + [pltpu.VMEM((4 * CHUNK,), jnp.float32)] * 2
        + [pltpu.VMEM((CHUNK, CP), jnp.float32)] * 8
        + [pltpu.VMEM((CHUNK * C,), jnp.float32)] * 2
        + [pltpu.VMEM((CHUNK * C,), jnp.float32)]
        + [pltpu.SMEM((2,), jnp.int32)]
        + [pltpu.SemaphoreType.DMA] * 4
    ),
)(_tile_body)


def kernel(local, image):
    coef = (local.astype(jnp.float32).reshape(B * 6)[:, None]
            + jnp.zeros((B * 6, L), jnp.float32))
    table = jnp.pad(image.reshape(B * HW, C), ((0, 0), (0, CP - C)))
    out = _sc_call(coef, table)
    return out.reshape(B, H, W, C)


# speedup vs baseline: 1.0377x; 1.0377x over previous
import functools

import jax
import jax.numpy as jnp
import numpy as np
from jax import lax
from jax.experimental import pallas as pl
from jax.experimental.pallas import tpu as pltpu
from jax.experimental.pallas import tpu_sc as plsc

B, H, W, C = 16, 224, 224, 96
HW = H * W
L = 16
NTILES = 32
CHUNK = 112
CPI = HW // CHUNK
NCHUNK = B * CPI // NTILES
NGRP = CHUNK // L
CROW = C // L
DELTA = float(np.float32(2.0) / np.float32(W - 1))


def _bf16r(x):
    u = lax.bitcast_convert_type(x, jnp.int32)
    u = u + 0x7FFF + ((u >> 16) & 1)
    u = lax.bitwise_and(u, jnp.int32(-65536))
    return lax.bitcast_convert_type(u, jnp.float32)


def _tile_body(coef_hbm, table_hbm, out_hbm, coef_v,
               ia0, ib0, ic0, id0, ia1, ib1, ic1, id1,
               w0, w1, ga0, gb0, gc0, gd0, ga1, gb1, gc1, gd1,
               o0, o1, zb, flag_v, gsem0, gsem1, osem0, osem1, zsem):
    idx_refs = ((ia0, ib0, ic0, id0), (ia1, ib1, ic1, id1))
    g_refs = ((ga0, gb0, gc0, gd0), (ga1, gb1, gc1, gd1))
    w_refs = (w0, w1)
    o_refs = (o0, o1)
    gsems = (gsem0, gsem1)
    osems = (osem0, osem1)
    wid = lax.axis_index("s") * 2 + lax.axis_index("c")

    pltpu.sync_copy(coef_hbm, coef_v)
    lane = lax.broadcasted_iota(jnp.int32, (L,), 0)

    def zero_grp(g, _):
        z = jnp.zeros((L,), jnp.float32)
        for r in range(CROW):
            zb[pl.ds(g * C + r * L, L)] = z
        return _

    lax.fori_loop(0, CHUNK, zero_grp, None, unroll=False)

    def compute_idx(cg, s):
        ia_v, ib_v, ic_v, id_v = idx_refs[s]
        w_v = w_refs[s]
        b = cg // CPI
        ci = cg % CPI
        imgbase = b * HW
        fi = (ci // 2).astype(jnp.float32)
        j0 = (ci % 2) * CHUNK
        ax = _bf16r(coef_v[b * 6 + 0])
        bx = _bf16r(coef_v[b * 6 + 1])
        cx = _bf16r(coef_v[b * 6 + 2])
        ay = _bf16r(coef_v[b * 6 + 3])
        by = _bf16r(coef_v[b * 6 + 4])
        cy = _bf16r(coef_v[b * 6 + 5])

        def idx_grp(g, acc):
            fj = (j0 + g * L + lane).astype(jnp.float32)
            xtv = _bf16r(fj * DELTA + (-1.0))
            ytv = _bf16r(jnp.zeros((L,), jnp.float32) + (fi * DELTA + (-1.0)))
            sgx = ax * xtv + bx * ytv + cx
            sgy = ay * xtv + by * ytv + cy
            xf = (sgx + 1.0) * (W * 0.5)
            yf = (sgy + 1.0) * (H * 0.5)
            xt = xf.astype(jnp.int32).astype(jnp.float32)
            x0f = jnp.where(xt > xf, xt - 1.0, xt)
            yt = yf.astype(jnp.int32).astype(jnp.float32)
            y0f = jnp.where(yt > yf, yt - 1.0, yt)
            x0c = jnp.clip(x0f, 0.0, W - 1.0)
            x1c = jnp.clip(x0f + 1.0, 0.0, W - 1.0)
            y0c = jnp.clip(y0f, 0.0, H - 1.0)
            y1c = jnp.clip(y0f + 1.0, 0.0, H - 1.0)
            dx1 = x1c - xf
            dx0 = xf - x0c
            dy1 = y1c - yf
            dy0 = yf - y0c
            x0i = x0c.astype(jnp.int32)
            y0i = y0c.astype(jnp.int32)
            dxs = x1c.astype(jnp.int32) - x0i
            dys = (y1c.astype(jnp.int32) - y0i) * W
            ia = imgbase + y0i * W + x0i
            sl = pl.ds(g * L, L)
            ia_v[sl] = ia
            ib_v[sl] = ia + dys
            ic_v[sl] = ia + dxs
            id_v[sl] = ia + dys + dxs
            w_v[pl.ds(0 * CHUNK + g * L, L)] = dx1 * dy1
            w_v[pl.ds(1 * CHUNK + g * L, L)] = dx1 * dy0
            w_v[pl.ds(2 * CHUNK + g * L, L)] = dx0 * dy1
            w_v[pl.ds(3 * CHUNK + g * L, L)] = dx0 * dy0
            okx = jnp.logical_and(xf >= 0.0, xf < W - 1.0)
            oky = jnp.logical_and(yf >= 0.0, yf < H - 1.0)
            ok = jnp.where(jnp.logical_and(okx, oky), 1.0, 0.0)
            return jnp.maximum(acc, ok)

        acc = lax.fori_loop(0, NGRP, idx_grp, jnp.zeros((L,), jnp.float32),
                            unroll=False)
        tot = acc[0]
        for p in range(1, L):
            tot = jnp.maximum(tot, acc[p])
        flag_v[s] = (tot > 0.0).astype(jnp.int32)

    def gather_copies(s):
        return [
            pltpu.make_async_copy(table_hbm.at[idx_refs[s][t]],
                                  g_refs[s][t], gsems[s])
            for t in range(4)
        ]

    def out_copy(cg, s):
        return pltpu.make_async_copy(
            o_refs[s], out_hbm.at[pl.ds(cg * (CHUNK * C), CHUNK * C)], osems[s])

    def zero_copy(cg, s):
        return pltpu.make_async_copy(
            zb, out_hbm.at[pl.ds(cg * (CHUNK * C), CHUNK * C)], osems[s])

    def blend(s):
        g_a, g_b, g_c, g_d = g_refs[s]
        w_v = w_refs[s]
        o_v = o_refs[s]

        def blend_grp(g, _):
            wrows = [w_v[pl.ds(t * CHUNK + g * L, L)] for t in range(4)]
            for p in range(L):
                pp = g * L + p
                wv = [jnp.zeros((L,), jnp.float32) + wrows[t][p]
                      for t in range(4)]
                for r in range(CROW):
                    cs = pl.ds(r * L, L)
                    acc = wv[0] * g_a[pp, cs] + wv[1] * g_b[pp, cs]
                    acc = acc + wv[2] * g_c[pp, cs]
                    acc = acc + wv[3] * g_d[pp, cs]
                    o_v[pl.ds(pp * C + r * L, L)] = acc
            return _

        lax.fori_loop(0, NGRP, blend_grp, None, unroll=False)

    def chunk_of(t):
        return wid * 2 + (t // 2) * (2 * NTILES) + (t % 2)

    compute_idx(chunk_of(jnp.int32(0)), 0)

    @pl.when(flag_v[0] != 0)
    def _fire0():
        for cp in gather_copies(0):
            cp.start()

    def pair_step(k, _):
        for s in range(2):
            t = 2 * k + s
            nxt = 1 - s

            @pl.when(t + 1 < NCHUNK)
            def _fire_next():
                compute_idx(chunk_of(t + 1), nxt)

                @pl.when(flag_v[nxt] != 0)
                def _start_next():
                    for cp in gather_copies(nxt):
                        cp.start()

            valid = flag_v[s] != 0

            @pl.when(valid)
            def _drain_gathers():
                for cp in gather_copies(s):
                    cp.wait()

            @pl.when(t >= 2)
            def _drain_out():
                out_copy(chunk_of(t - 2), s).wait()

            @pl.when(valid)
            def _do_blend():
                blend(s)
                out_copy(chunk_of(t), s).start()

            @pl.when(jnp.logical_not(valid))
            def _do_zero():
                zero_copy(chunk_of(t), s).start()
        return _

    lax.fori_loop(0, NCHUNK // 2, pair_step, None, unroll=False)
    out_copy(chunk_of(jnp.int32(NCHUNK - 2)), 0).wait()
    out_copy(chunk_of(jnp.int32(NCHUNK - 1)), 1).wait()


_sc_call = functools.partial(
    pl.kernel,
    mesh=plsc.VectorSubcoreMesh(core_axis_name="c", subcore_axis_name="s"),
    out_type=jax.ShapeDtypeStruct((B * HW * C,), jnp.float32),
    compiler_params=pltpu.CompilerParams(use_tc_tiling_on_sc=False),
    scratch_types=(
        [pltpu.VMEM((B * 6, L), jnp.float32)]
        + [pltpu.VMEM((CHUNK,), jnp.int32)] * 8
        + [pltpu.VMEM((4 * CHUNK,), jnp.float32)] * 2
        + [pltpu.VMEM((CHUNK, C), jnp.float32)] * 8
        + [pltpu.VMEM((CHUNK * C,), jnp.float32)] * 2
        + [pltpu.VMEM((CHUNK * C,), jnp.float32)]
        + [pltpu.SMEM((2,), jnp.int32)]
        + [pltpu.SemaphoreType.DMA] * 5
    ),
)(_tile_body)


def kernel(local, image):
    coef = (local.astype(jnp.float32).reshape(B * 6)[:, None]
            + jnp.zeros((B * 6, L), jnp.float32))
    table = image.reshape(B * HW, C)
    out = _sc_call(coef, table)
    return out.reshape(B, H, W, C)
